# Initial kernel scaffold; baseline (speedup 1.0000x reference)
import jax
import jax.numpy as jnp
from jax.experimental import pallas as pl


def kernel(x, table):
    raise NotImplementedError("write your pallas kernel here")


# speedup vs baseline: 1.8746x; 1.8746x over previous
import functools

import jax
import jax.numpy as jnp
from jax import lax
from jax.experimental import pallas as pl
from jax.experimental.pallas import tpu as pltpu
from jax.experimental.pallas import tpu_sc as plsc

_CHUNK = 128
_NBUF = 8
_PF = 4


@functools.lru_cache(maxsize=None)
def _make_gather(n_idx, vocab, d):
    info = plsc.get_sparse_core_info()
    nw = info.num_cores * info.num_subcores
    b_per_w = n_idx // nw
    n_chunks = b_per_w // _CHUNK
    n_grp = n_chunks // _NBUF
    assert n_chunks * _CHUNK * nw == n_idx and n_grp * _NBUF == n_chunks

    mesh = plsc.VectorSubcoreMesh(core_axis_name="c", subcore_axis_name="s")

    @functools.partial(
        pl.kernel,
        out_type=jax.ShapeDtypeStruct((n_idx, d), jnp.float32),
        mesh=mesh,
        compiler_params=pltpu.CompilerParams(use_tc_tiling_on_sc=False),
        scratch_types=[
            pltpu.VMEM((n_chunks, _CHUNK), jnp.int32),
            pltpu.VMEM((_NBUF, _CHUNK, d), jnp.float32),
            pltpu.SemaphoreType.DMA((_NBUF,)),
            pltpu.SemaphoreType.DMA((_NBUF,)),
        ],
    )
    def gather_kernel(idx_hbm, table_hbm, out_hbm, idx_v, rows_v, gsem, ssem):
        wid = lax.axis_index("s") * info.num_cores + lax.axis_index("c")
        cbase = wid * n_chunks
        pltpu.sync_copy(idx_hbm.at[pl.ds(cbase, n_chunks)], idx_v)

        def start_gather(j, b):
            pltpu.async_copy(table_hbm.at[idx_v.at[j]], rows_v.at[b], gsem.at[b])

        def wait_gather(j, b):
            pltpu.make_async_copy(
                table_hbm.at[idx_v.at[j]], rows_v.at[b], gsem.at[b]
            ).wait()

        def start_store(j, b):
            pltpu.async_copy(
                rows_v.at[b], out_hbm.at[pl.ds((cbase + j) * _CHUNK, _CHUNK)],
                ssem.at[b],
            )

        def wait_store(b):
            pltpu.make_async_copy(
                rows_v.at[b], out_hbm.at[pl.ds(cbase * _CHUNK, _CHUNK)],
                ssem.at[b],
            ).wait()

        for b in range(_PF):
            start_gather(b, b)

        def body(g, carry):
            j0 = g * _NBUF
            for b in range(_NBUF):
                j = j0 + b
                wait_gather(j, b)
                start_store(j, b)
                jp = j + _PF
                bp = (b + _PF) % _NBUF

                @pl.when(jp < n_chunks)
                def _():
                    @pl.when(jp >= _NBUF)
                    def _():
                        wait_store(bp)

                    start_gather(jp, bp)

            return carry

        lax.fori_loop(0, n_grp, body, 0)
        for b in range(_NBUF):
            wait_store(b)

    return gather_kernel


def kernel(x, table):
    batch, hist = x.shape
    vocab, d = table.shape
    n_idx = batch * hist
    idx = x.reshape(n_idx // _CHUNK, _CHUNK).astype(jnp.int32)
    out = _make_gather(n_idx, vocab, d)(idx, table)
    return out.reshape(batch, hist, d)


# speedup vs baseline: 2.1770x; 1.1614x over previous
import functools

import jax
import jax.numpy as jnp
from jax import lax
from jax.experimental import pallas as pl
from jax.experimental.pallas import tpu as pltpu
from jax.experimental.pallas import tpu_sc as plsc
from jax.experimental.layout import Layout, with_layout_constraint

_CHUNK = 128
_NBUF = 8
_PF = 4


@functools.lru_cache(maxsize=None)
def _make_gather(n_idx, vocab, d):
    info = plsc.get_sparse_core_info()
    nw = info.num_cores * info.num_subcores
    b_per_w = n_idx // nw
    n_chunks = b_per_w // _CHUNK
    n_grp = n_chunks // _NBUF
    assert n_chunks * _CHUNK * nw == n_idx and n_grp * _NBUF == n_chunks

    mesh = plsc.VectorSubcoreMesh(core_axis_name="c", subcore_axis_name="s")

    @functools.partial(
        pl.kernel,
        out_type=jax.ShapeDtypeStruct((n_idx, d), jnp.float32),
        mesh=mesh,
        compiler_params=pltpu.CompilerParams(use_tc_tiling_on_sc=False),
        scratch_types=[
            pltpu.VMEM((n_chunks, _CHUNK), jnp.int32),
            pltpu.VMEM((_NBUF, _CHUNK, d), jnp.float32),
            pltpu.SemaphoreType.DMA((_NBUF,)),
            pltpu.SemaphoreType.DMA((_NBUF,)),
        ],
    )
    def gather_kernel(idx_hbm, table_hbm, out_hbm, idx_v, rows_v, gsem, ssem):
        wid = lax.axis_index("s") * info.num_cores + lax.axis_index("c")
        cbase = wid * n_chunks
        pltpu.sync_copy(idx_hbm.at[pl.ds(cbase, n_chunks)], idx_v)

        def start_gather(j, b):
            pltpu.async_copy(table_hbm.at[idx_v.at[j]], rows_v.at[b], gsem.at[b])

        def wait_gather(j, b):
            pltpu.make_async_copy(
                table_hbm.at[idx_v.at[j]], rows_v.at[b], gsem.at[b]
            ).wait()

        def start_store(j, b):
            pltpu.async_copy(
                rows_v.at[b], out_hbm.at[pl.ds((cbase + j) * _CHUNK, _CHUNK)],
                ssem.at[b],
            )

        def wait_store(b):
            pltpu.make_async_copy(
                rows_v.at[b], out_hbm.at[pl.ds(cbase * _CHUNK, _CHUNK)],
                ssem.at[b],
            ).wait()

        for b in range(_PF):
            start_gather(b, b)

        def body(g, carry):
            j0 = g * _NBUF
            for b in range(_NBUF):
                j = j0 + b
                wait_gather(j, b)
                start_store(j, b)
                jp = j + _PF
                bp = (b + _PF) % _NBUF

                @pl.when(jp < n_chunks)
                def _():
                    @pl.when(jp >= _NBUF)
                    def _():
                        wait_store(bp)

                    start_gather(jp, bp)

            return carry

        lax.fori_loop(0, n_grp, body, 0)
        for b in range(_NBUF):
            wait_store(b)

    return gather_kernel


def kernel(x, table):
    batch, hist = x.shape
    vocab, d = table.shape
    n_idx = batch * hist
    idx = x.reshape(n_idx // _CHUNK, _CHUNK).astype(jnp.int32)
    out = _make_gather(n_idx, vocab, d)(idx, table)
    out3 = out.reshape(batch, hist, d)
    return with_layout_constraint(out3, Layout((0, 1, 2)))
